# Initial kernel scaffold; baseline (speedup 1.0000x reference)
import jax
import jax.numpy as jnp
from jax.experimental import pallas as pl


def kernel(x_s, edge_index_s, edge_attr_s, x_s_batch, x_t, edge_index_t, edge_attr_t, x_t_batch):
    raise NotImplementedError("write your pallas kernel here")


# speedup vs baseline: 196.5354x; 196.5354x over previous
import functools

import jax
import jax.numpy as jnp
from jax import lax
from jax.experimental import pallas as pl
from jax.experimental.pallas import tpu as pltpu
from jax.experimental.pallas import tpu_sc as plsc

NS = 50000
NPT = 3200
HALF = 16 * NPT
NP = 2 * HALF
E1 = 1600000
EPT = E1 // 16
EC = 2000
NCH = EPT // EC
VI = NPT // 16
M = 1 << 19
TBL = M + 512
TSL = TBL // 16

_U = jnp.uint32


def _mix(c, seed):
    c = c + _U(seed)
    c = c * _U(0x9E3779B1)
    c = c ^ (c >> _U(15))
    c = c * _U(0x85EBCA6B)
    c = c ^ (c >> _U(13))
    c = c * _U(0xC2B2AE35)
    c = c ^ (c >> _U(16))
    return c


def _u(x):
    return lax.bitcast_convert_type(x, jnp.uint32)


def _i(x):
    return lax.bitcast_convert_type(x, jnp.int32)


_MESH = plsc.VectorSubcoreMesh(core_axis_name="c", subcore_axis_name="s",
                               num_cores=2, num_subcores=16)
_IOTA = lambda: lax.iota(jnp.int32, 16)


def _wid():
    c = lax.axis_index("c")
    s = lax.axis_index("s")
    return c, s, c * 16 + s


def _node_phase(init):
    def body(*refs):
        if init:
            (x_hbm, ca_out, cb_out, gs_out, cs_out, ct_out,
             ab, bb, h1b, h2b, gsb, ssb, onesb, tbuf, cnt_sp) = refs
        else:
            (ca_hbm, cb_hbm, h1_hbm, h2_hbm, ca_out, cb_out, gs_out,
             cs_out, ct_out, ab, bb, h1b, h2b, gsb, ssb, onesb, tbuf,
             cnt_sp) = refs
        c, s, wid = _wid()
        base = c * HALF + s * NPT

        def zstep(i, _):
            gsb[pl.ds(i * 16, 16)] = jnp.zeros((16,), jnp.int32)
            return _

        lax.fori_loop(0, VI, zstep, 0)
        for k in range(TSL // NPT):
            pltpu.sync_copy(gsb, cnt_sp.at[pl.ds(s * TSL + k * NPT, NPT)])
        rem = TSL % NPT
        if rem:
            pltpu.sync_copy(gsb.at[pl.ds(0, rem)],
                            cnt_sp.at[pl.ds(s * TSL + TSL - rem, rem)])
        if init:
            pltpu.sync_copy(x_hbm.at[pl.ds(base, NPT)], ab)
        else:
            pltpu.sync_copy(ca_hbm.at[pl.ds(base, NPT)], ab)
            pltpu.sync_copy(cb_hbm.at[pl.ds(base, NPT)], bb)
            pltpu.sync_copy(h1_hbm.at[c, pl.ds(s * NPT, NPT)], h1b)
            pltpu.sync_copy(h2_hbm.at[c, pl.ds(s * NPT, NPT)], h2b)

        def step(i, _):
            sl = pl.ds(i * 16, 16)
            if init:
                xv = _u(ab[sl])
                a = _mix(xv, 0x165667B1)
                b = _mix(xv, 0x85EBCA77)
            else:
                h1 = _u(h1b[sl])
                h2 = _u(h2b[sl])
                a = _mix(_mix(_u(ab[sl]) ^ h1, 0x9E3779B9) ^ h2, 0xC2B2AE3D)
                b = _mix(_mix(_u(bb[sl]) ^ h1, 0x68E31DA4) ^ h2, 0xB5297A4D)
            slot = _i(_mix(a ^ (b * _U(0x27D4EB2F)), 0xB5297A4D) & _U(M - 1))
            idxl = s * NPT + i * 16 + _IOTA()
            real = idxl < NS
            ssb[sl] = jnp.where(real, slot, M + 256 + wid)
            gsb[sl] = jnp.where(real, slot, M)
            ab[sl] = _i(a)
            bb[sl] = _i(b)
            onesb[sl] = jnp.full((16,), 1, jnp.int32)
            return _

        lax.fori_loop(0, VI, step, 0)
        plsc.subcore_barrier()
        pltpu.sync_copy(onesb, cnt_sp.at[ssb], add=True)
        pltpu.sync_copy(ab, ca_out.at[pl.ds(base, NPT)])
        pltpu.sync_copy(bb, cb_out.at[pl.ds(base, NPT)])
        pltpu.sync_copy(gsb, gs_out.at[pl.ds(base, NPT)])
        plsc.subcore_barrier()
        tsl = pl.ds(s * TSL, TSL)
        pltpu.sync_copy(cnt_sp.at[tsl], tbuf)

        @pl.when(c == 0)
        def _():
            pltpu.sync_copy(tbuf, cs_out.at[tsl])

        @pl.when(c == 1)
        def _():
            pltpu.sync_copy(tbuf, ct_out.at[tsl])

    sds = jax.ShapeDtypeStruct
    return pl.kernel(
        body,
        out_type=[sds((NP,), jnp.int32), sds((NP,), jnp.int32),
                  sds((NP,), jnp.int32), sds((TBL,), jnp.int32),
                  sds((TBL,), jnp.int32)],
        mesh=_MESH,
        scratch_types=[pltpu.VMEM((NPT,), jnp.int32)] * 7
        + [pltpu.VMEM((TSL,), jnp.int32)]
        + [pltpu.VMEM_SHARED((TBL,), jnp.int32)],
    )


def _edge_phase():
    def body(ca_hbm, src_hbm, dst_hbm, h1_out, h2_out,
             srcb, dstb, gb, f1b, f2b, nbuf, ca_sp, h1_sp, h2_sp, sem):
        c, s, wid = _wid()
        nsl = pl.ds(s * NPT, NPT)
        pltpu.sync_copy(ca_hbm.at[pl.ds(c * HALF + s * NPT, NPT)], nbuf)
        pltpu.sync_copy(nbuf, ca_sp.at[nsl])

        def zstep(i, _):
            nbuf[pl.ds(i * 16, 16)] = jnp.zeros((16,), jnp.int32)
            return _

        lax.fori_loop(0, VI, zstep, 0)
        pltpu.sync_copy(nbuf, h1_sp.at[nsl])
        pltpu.sync_copy(nbuf, h2_sp.at[nsl])
        plsc.subcore_barrier()
        ebase = c * E1 + s * EPT

        def chunk(k, _):
            esl = pl.ds(ebase + k * EC, EC)
            pltpu.sync_copy(src_hbm.at[esl], srcb)
            pltpu.sync_copy(dst_hbm.at[esl], dstb)
            pltpu.async_copy(ca_sp.at[srcb], gb, sem).wait()

            def step(j, _):
                sl = pl.ds(j * 16, 16)
                cav = _u(gb[sl])
                f1b[sl] = _i(_mix(cav, 1013904223))
                f2b[sl] = _i(_mix(cav, 374761393))
                return _

            lax.fori_loop(0, EC // 16, step, 0)
            pltpu.sync_copy(f1b, h1_sp.at[dstb], add=True)
            pltpu.sync_copy(f2b, h2_sp.at[dstb], add=True)
            return _

        lax.fori_loop(0, NCH, chunk, 0)
        plsc.subcore_barrier()
        pltpu.sync_copy(h1_sp.at[nsl], nbuf)
        pltpu.sync_copy(nbuf, h1_out.at[c, nsl])
        pltpu.sync_copy(h2_sp.at[nsl], nbuf)
        pltpu.sync_copy(nbuf, h2_out.at[c, nsl])

    sds = jax.ShapeDtypeStruct
    return pl.kernel(
        body,
        out_type=[sds((2, HALF), jnp.int32), sds((2, HALF), jnp.int32)],
        mesh=_MESH,
        scratch_types=[pltpu.VMEM((EC,), jnp.int32)] * 5
        + [pltpu.VMEM((NPT,), jnp.int32)]
        + [pltpu.VMEM_SHARED((HALF,), jnp.int32)] * 3
        + [pltpu.SemaphoreType.DMA],
    )


def _stats_phase():
    def body(gs_hbm, cs_hbm, ct_hbm, out_hbm, gsb, vsb, vtb, sb, sem):
        c, s, wid = _wid()
        base = c * HALF + s * NPT
        pltpu.sync_copy(gs_hbm.at[pl.ds(base, NPT)], gsb)
        pltpu.async_copy(cs_hbm.at[gsb], vsb, sem).wait()
        pltpu.async_copy(ct_hbm.at[gsb], vtb, sem).wait()

        def step(i, acc):
            sl = pl.ds(i * 16, 16)
            return (acc[0] + vsb[sl].astype(jnp.float32),
                    acc[1] + vtb[sl].astype(jnp.float32))

        z = jnp.zeros((16,), jnp.float32)
        acc_a, acc_b = lax.fori_loop(0, VI, step, (z, z))
        sb[pl.ds(0, 16)] = acc_a
        sb[pl.ds(16, 16)] = acc_b
        pltpu.sync_copy(sb, out_hbm.at[wid])

    sds = jax.ShapeDtypeStruct
    return pl.kernel(
        body,
        out_type=sds((32, 32), jnp.float32),
        mesh=_MESH,
        scratch_types=[pltpu.VMEM((NPT,), jnp.int32)] * 3
        + [pltpu.VMEM((32,), jnp.float32), pltpu.SemaphoreType.DMA],
    )


_init_k = _node_phase(init=True)
_node_k = _node_phase(init=False)
_edge_k = _edge_phase()
_stats_k = _stats_phase()

NUM_ROUNDS = 3


def kernel(x_s, edge_index_s, edge_attr_s, x_s_batch,
           x_t, edge_index_t, edge_attr_t, x_t_batch):
    pad = jnp.zeros((HALF - NS,), jnp.int32)
    x = jnp.concatenate([x_s.astype(jnp.int32), pad,
                         x_t.astype(jnp.int32), pad])
    src = jnp.concatenate([edge_index_s[0], edge_index_t[0]]).astype(jnp.int32)
    dst = jnp.concatenate([edge_index_s[1], edge_index_t[1]]).astype(jnp.int32)
    ca, cb, gs, cs, ct = _init_k(x)
    parts = [_stats_k(gs, cs, ct)]
    for _ in range(NUM_ROUNDS):
        h1, h2 = _edge_k(ca, src, dst)
        ca, cb, gs, cs, ct = _node_k(ca, cb, h1, h2)
        parts.append(_stats_k(gs, cs, ct))

    p = jnp.stack(parts)
    ssum = jnp.sum(p[:, :16, :16])
    dsum = jnp.sum(p[:, :16, 16:])
    tsum = jnp.sum(p[:, 16:, 16:])
    den = (jnp.maximum(jnp.sqrt(ssum), 1e-8)
           * jnp.maximum(jnp.sqrt(tsum), 1e-8))
    return jnp.reshape(dsum / den, (1,))


# speedup vs baseline: 222.5833x; 1.1325x over previous
import functools

import jax
import jax.numpy as jnp
from jax import lax
from jax.experimental import pallas as pl
from jax.experimental.pallas import tpu as pltpu
from jax.experimental.pallas import tpu_sc as plsc

NS = 50000
NPT = 3200
HALF = 16 * NPT
NP = 2 * HALF
E1 = 1600000
EPT = E1 // 16
EC = 10000
NCH = EPT // EC
VI = NPT // 16
M = 1 << 19
TBL = M + 512
TSL = TBL // 16

_U = jnp.uint32


def _mix(c, seed):
    c = c + _U(seed)
    c = c * _U(0x9E3779B1)
    c = c ^ (c >> _U(15))
    c = c * _U(0x85EBCA6B)
    c = c ^ (c >> _U(13))
    c = c * _U(0xC2B2AE35)
    c = c ^ (c >> _U(16))
    return c


def _u(x):
    return lax.bitcast_convert_type(x, jnp.uint32)


def _i(x):
    return lax.bitcast_convert_type(x, jnp.int32)


_MESH = plsc.VectorSubcoreMesh(core_axis_name="c", subcore_axis_name="s",
                               num_cores=2, num_subcores=16)
_IOTA = lambda: lax.iota(jnp.int32, 16)


def _wid():
    c = lax.axis_index("c")
    s = lax.axis_index("s")
    return c, s, c * 16 + s


def _node_phase(init):
    def body(*refs):
        if init:
            (x_hbm, ca_out, cb_out, gs_out, cs_out, ct_out,
             ab, bb, h1b, tb, gsb, ssb, onesb, tbuf, cnt_sp) = refs
        else:
            (ca_hbm, cb_hbm, hp_hbm, ca_out, cb_out, gs_out,
             cs_out, ct_out, ab, bb, h1b, tb, gsb, ssb, onesb, tbuf,
             cnt_sp) = refs
        c, s, wid = _wid()
        base = c * HALF + s * NPT

        def zstep(i, _):
            gsb[pl.ds(i * 16, 16)] = jnp.zeros((16,), jnp.int32)
            return _

        lax.fori_loop(0, VI, zstep, 0)
        for k in range(TSL // NPT):
            pltpu.sync_copy(gsb, cnt_sp.at[pl.ds(s * TSL + k * NPT, NPT)])
        rem = TSL % NPT
        if rem:
            pltpu.sync_copy(gsb.at[pl.ds(0, rem)],
                            cnt_sp.at[pl.ds(s * TSL + TSL - rem, rem)])
        if init:
            pltpu.sync_copy(x_hbm.at[pl.ds(base, NPT)], ab)
        else:
            pltpu.sync_copy(ca_hbm.at[pl.ds(base, NPT)], ab)
            pltpu.sync_copy(cb_hbm.at[pl.ds(base, NPT)], bb)
            nsl = pl.ds(s * NPT, NPT)
            pltpu.sync_copy(hp_hbm.at[c * 16, nsl], h1b)

            def racc(t, _):
                pltpu.sync_copy(hp_hbm.at[c * 16 + t, nsl], tb)

                def astep(i, _):
                    sl = pl.ds(i * 16, 16)
                    h1b[sl] = h1b[sl] + tb[sl]
                    return _

                lax.fori_loop(0, VI, astep, 0)
                return _

            lax.fori_loop(1, 16, racc, 0)

        def step(i, _):
            sl = pl.ds(i * 16, 16)
            if init:
                xv = _u(ab[sl])
                a = _mix(xv, 0x165667B1)
                b = _mix(xv, 0x85EBCA77)
            else:
                m1 = _mix(_u(h1b[sl]), 0x27D4EB2F)
                a = _mix(_u(ab[sl]) ^ m1, 0x9E3779B9)
                b = _mix(_u(bb[sl]) ^ (m1 * _U(0x165667B1)), 0x68E31DA4)
            slot = _i(_mix(a ^ (b * _U(0x27D4EB2F)), 0xB5297A4D) & _U(M - 1))
            idxl = s * NPT + i * 16 + _IOTA()
            real = idxl < NS
            ssb[sl] = jnp.where(real, slot, M + 256 + wid)
            gsb[sl] = jnp.where(real, slot, M)
            ab[sl] = _i(a)
            bb[sl] = _i(b)
            onesb[sl] = jnp.full((16,), 1, jnp.int32)
            return _

        lax.fori_loop(0, VI, step, 0)
        plsc.subcore_barrier()
        pltpu.sync_copy(onesb, cnt_sp.at[ssb], add=True)
        pltpu.sync_copy(ab, ca_out.at[pl.ds(base, NPT)])
        pltpu.sync_copy(bb, cb_out.at[pl.ds(base, NPT)])
        pltpu.sync_copy(gsb, gs_out.at[pl.ds(base, NPT)])
        plsc.subcore_barrier()
        tsl = pl.ds(s * TSL, TSL)
        pltpu.sync_copy(cnt_sp.at[tsl], tbuf)

        @pl.when(c == 0)
        def _():
            pltpu.sync_copy(tbuf, cs_out.at[tsl])

        @pl.when(c == 1)
        def _():
            pltpu.sync_copy(tbuf, ct_out.at[tsl])

    sds = jax.ShapeDtypeStruct
    return pl.kernel(
        body,
        out_type=[sds((NP,), jnp.int32), sds((NP,), jnp.int32),
                  sds((NP,), jnp.int32), sds((TBL,), jnp.int32),
                  sds((TBL,), jnp.int32)],
        mesh=_MESH,
        scratch_types=[pltpu.VMEM((NPT,), jnp.int32)] * 7
        + [pltpu.VMEM((TSL,), jnp.int32)]
        + [pltpu.VMEM_SHARED((TBL,), jnp.int32)],
    )


def _edge_phase():
    def body(ca_hbm, src_hbm, dst_hbm, hp_out, cab, h1b, srcb, dstb):
        c, s, wid = _wid()
        pltpu.sync_copy(ca_hbm.at[pl.ds(c * HALF, HALF)], cab)
        z = jnp.zeros((16,), jnp.int32)

        def zstep(i, _):
            b = i * 64
            h1b[pl.ds(b, 16)] = z
            h1b[pl.ds(b + 16, 16)] = z
            h1b[pl.ds(b + 32, 16)] = z
            h1b[pl.ds(b + 48, 16)] = z
            return _

        lax.fori_loop(0, HALF // 64, zstep, 0)
        ebase = c * E1 + s * EPT

        def chunk(k, _):
            esl = pl.ds(ebase + k * EC, EC)
            pltpu.sync_copy(src_hbm.at[esl], srcb)
            pltpu.sync_copy(dst_hbm.at[esl], dstb)

            def step(j, _):
                sl = pl.ds(j * 16, 16)
                cav = _u(plsc.load_gather(cab, [srcb[sl]]))
                f1 = _mix(cav, 1013904223)
                plsc.addupdate_scatter(h1b, [dstb[sl]], _i(f1))
                return _

            lax.fori_loop(0, EC // 16, step, 0)
            return _

        lax.fori_loop(0, NCH, chunk, 0)
        pltpu.sync_copy(h1b, hp_out.at[wid])

    sds = jax.ShapeDtypeStruct
    return pl.kernel(
        body,
        out_type=sds((32, HALF), jnp.int32),
        mesh=_MESH,
        scratch_types=[pltpu.VMEM((HALF,), jnp.int32)] * 2
        + [pltpu.VMEM((EC,), jnp.int32)] * 2,
        compiler_params=pltpu.CompilerParams(use_tc_tiling_on_sc=False,
                                             needs_layout_passes=False),
    )


def _stats_phase():
    def body(gs_hbm, cs_hbm, ct_hbm, out_hbm, gsb, vsb, vtb, sb, sem):
        c, s, wid = _wid()
        base = c * HALF + s * NPT
        pltpu.sync_copy(gs_hbm.at[pl.ds(base, NPT)], gsb)
        pltpu.async_copy(cs_hbm.at[gsb], vsb, sem).wait()
        pltpu.async_copy(ct_hbm.at[gsb], vtb, sem).wait()

        def step(i, acc):
            sl = pl.ds(i * 16, 16)
            return (acc[0] + vsb[sl].astype(jnp.float32),
                    acc[1] + vtb[sl].astype(jnp.float32))

        z = jnp.zeros((16,), jnp.float32)
        acc_a, acc_b = lax.fori_loop(0, VI, step, (z, z))
        sb[pl.ds(0, 16)] = acc_a
        sb[pl.ds(16, 16)] = acc_b
        pltpu.sync_copy(sb, out_hbm.at[wid])

    sds = jax.ShapeDtypeStruct
    return pl.kernel(
        body,
        out_type=sds((32, 32), jnp.float32),
        mesh=_MESH,
        scratch_types=[pltpu.VMEM((NPT,), jnp.int32)] * 3
        + [pltpu.VMEM((32,), jnp.float32), pltpu.SemaphoreType.DMA],
    )


_init_k = _node_phase(init=True)
_node_k = _node_phase(init=False)
_edge_k = _edge_phase()
_stats_k = _stats_phase()

NUM_ROUNDS = 3


def kernel(x_s, edge_index_s, edge_attr_s, x_s_batch,
           x_t, edge_index_t, edge_attr_t, x_t_batch):
    pad = jnp.zeros((HALF - NS,), jnp.int32)
    x = jnp.concatenate([x_s.astype(jnp.int32), pad,
                         x_t.astype(jnp.int32), pad])
    src = jnp.concatenate([edge_index_s[0], edge_index_t[0]]).astype(jnp.int32)
    dst = jnp.concatenate([edge_index_s[1], edge_index_t[1]]).astype(jnp.int32)
    ca, cb, gs, cs, ct = _init_k(x)
    parts = [_stats_k(gs, cs, ct)]
    for _ in range(NUM_ROUNDS):
        hp = _edge_k(ca, src, dst)
        ca, cb, gs, cs, ct = _node_k(ca, cb, hp)
        parts.append(_stats_k(gs, cs, ct))

    p = jnp.stack(parts)
    ssum = jnp.sum(p[:, :16, :16])
    dsum = jnp.sum(p[:, :16, 16:])
    tsum = jnp.sum(p[:, 16:, 16:])
    den = (jnp.maximum(jnp.sqrt(ssum), 1e-8)
           * jnp.maximum(jnp.sqrt(tsum), 1e-8))
    return jnp.reshape(dsum / den, (1,))


# speedup vs baseline: 261.2205x; 1.1736x over previous
import functools

import jax
import jax.numpy as jnp
from jax import lax
from jax.experimental import pallas as pl
from jax.experimental.pallas import tpu as pltpu
from jax.experimental.pallas import tpu_sc as plsc

NS = 50000
NPT = 3200
HALF = 16 * NPT
NP = 2 * HALF
E1 = 1600000
EPT = E1 // 16
EC = 4000
NCH = EPT // EC
VI = NPT // 16
M = 1 << 19
TBL = M + 512
TSL = TBL // 16

_U = jnp.uint32


def _mix(c, seed):
    c = c + _U(seed)
    c = c * _U(0x9E3779B1)
    c = c ^ (c >> _U(15))
    c = c * _U(0x85EBCA6B)
    c = c ^ (c >> _U(13))
    c = c * _U(0xC2B2AE35)
    c = c ^ (c >> _U(16))
    return c


def _u(x):
    return lax.bitcast_convert_type(x, jnp.uint32)


def _i(x):
    return lax.bitcast_convert_type(x, jnp.int32)


_MESH = plsc.VectorSubcoreMesh(core_axis_name="c", subcore_axis_name="s",
                               num_cores=2, num_subcores=16)
_IOTA = lambda: lax.iota(jnp.int32, 16)


def _wid():
    c = lax.axis_index("c")
    s = lax.axis_index("s")
    return c, s, c * 16 + s


def _node_phase(init):
    def body(*refs):
        if init:
            (x_hbm, ca_out, cb_out, gs_out, cs_out, ct_out,
             ab, bb, h1b, tb0, tb1, gsb, ssb, onesb, tbuf, cnt_sp,
             sem0, sem1) = refs
        else:
            (ca_hbm, cb_hbm, hp_hbm, ca_out, cb_out, gs_out,
             cs_out, ct_out, ab, bb, h1b, tb0, tb1, gsb, ssb, onesb, tbuf,
             cnt_sp, sem0, sem1) = refs
        c, s, wid = _wid()
        base = c * HALF + s * NPT

        def zstep(i, _):
            gsb[pl.ds(i * 16, 16)] = jnp.zeros((16,), jnp.int32)
            return _

        lax.fori_loop(0, VI, zstep, 0)
        for k in range(TSL // NPT):
            pltpu.sync_copy(gsb, cnt_sp.at[pl.ds(s * TSL + k * NPT, NPT)])
        rem = TSL % NPT
        if rem:
            pltpu.sync_copy(gsb.at[pl.ds(0, rem)],
                            cnt_sp.at[pl.ds(s * TSL + TSL - rem, rem)])
        if init:
            pltpu.sync_copy(x_hbm.at[pl.ds(base, NPT)], ab)
        else:
            pltpu.sync_copy(ca_hbm.at[pl.ds(base, NPT)], ab)
            pltpu.sync_copy(cb_hbm.at[pl.ds(base, NPT)], bb)
            nsl = pl.ds(s * NPT, NPT)
            pltpu.sync_copy(hp_hbm.at[c * 16, nsl], h1b)
            bufs = ((tb0, sem0), (tb1, sem1))
            pend = pltpu.async_copy(hp_hbm.at[c * 16 + 1, nsl], tb0, sem0)
            for t in range(1, 16):
                tb, _sem = bufs[(t - 1) % 2]
                pend.wait()
                if t + 1 < 16:
                    nb, nsem = bufs[t % 2]
                    pend = pltpu.async_copy(hp_hbm.at[c * 16 + t + 1, nsl],
                                            nb, nsem)

                def astep(i, _, tb=tb):
                    b = i * 64
                    for q in range(4):
                        sl = pl.ds(b + q * 16, 16)
                        h1b[sl] = h1b[sl] + tb[sl]
                    return _

                lax.fori_loop(0, VI // 4, astep, 0)

        def step(i, _):
            sl = pl.ds(i * 16, 16)
            if init:
                xv = _u(ab[sl])
                a = _mix(xv, 0x165667B1)
                b = _mix(xv, 0x85EBCA77)
            else:
                m1 = _mix(_u(h1b[sl]), 0x27D4EB2F)
                a = _mix(_u(ab[sl]) ^ m1, 0x9E3779B9)
                b = _mix(_u(bb[sl]) ^ (m1 * _U(0x165667B1)), 0x68E31DA4)
            slot = _i(_mix(a ^ (b * _U(0x27D4EB2F)), 0xB5297A4D) & _U(M - 1))
            idxl = s * NPT + i * 16 + _IOTA()
            real = idxl < NS
            ssb[sl] = jnp.where(real, slot, M + 256 + wid)
            gsb[sl] = jnp.where(real, slot, M)
            ab[sl] = _i(a)
            bb[sl] = _i(b)
            onesb[sl] = jnp.full((16,), 1, jnp.int32)
            return _

        lax.fori_loop(0, VI, step, 0)
        plsc.subcore_barrier()
        pltpu.sync_copy(onesb, cnt_sp.at[ssb], add=True)
        pltpu.sync_copy(ab, ca_out.at[pl.ds(base, NPT)])
        pltpu.sync_copy(bb, cb_out.at[pl.ds(base, NPT)])
        pltpu.sync_copy(gsb, gs_out.at[pl.ds(base, NPT)])
        plsc.subcore_barrier()
        tsl = pl.ds(s * TSL, TSL)
        pltpu.sync_copy(cnt_sp.at[tsl], tbuf)

        @pl.when(c == 0)
        def _():
            pltpu.sync_copy(tbuf, cs_out.at[tsl])

        @pl.when(c == 1)
        def _():
            pltpu.sync_copy(tbuf, ct_out.at[tsl])

    sds = jax.ShapeDtypeStruct
    return pl.kernel(
        body,
        out_type=[sds((NP,), jnp.int32), sds((NP,), jnp.int32),
                  sds((NP,), jnp.int32), sds((TBL,), jnp.int32),
                  sds((TBL,), jnp.int32)],
        mesh=_MESH,
        scratch_types=[pltpu.VMEM((NPT,), jnp.int32)] * 8
        + [pltpu.VMEM((TSL,), jnp.int32)]
        + [pltpu.VMEM_SHARED((TBL,), jnp.int32)]
        + [pltpu.SemaphoreType.DMA] * 2,
    )


def _edge_phase():
    def body(ca_hbm, src_hbm, dst_hbm, hp_out,
             cab, h1b, srcb0, dstb0, srcb1, dstb1, sem0, sem1):
        c, s, wid = _wid()
        pltpu.sync_copy(ca_hbm.at[pl.ds(c * HALF, HALF)], cab)
        z = jnp.zeros((16,), jnp.int32)

        def zstep(i, _):
            b = i * 64
            h1b[pl.ds(b, 16)] = z
            h1b[pl.ds(b + 16, 16)] = z
            h1b[pl.ds(b + 32, 16)] = z
            h1b[pl.ds(b + 48, 16)] = z
            return _

        lax.fori_loop(0, HALF // 64, zstep, 0)
        ebase = c * E1 + s * EPT
        bufs = ((srcb0, dstb0, sem0), (srcb1, dstb1, sem1))

        def start(k):
            sb, db, sem = bufs[k % 2]
            esl = pl.ds(ebase + k * EC, EC)
            return (pltpu.async_copy(src_hbm.at[esl], sb, sem),
                    pltpu.async_copy(dst_hbm.at[esl], db, sem))

        pend = start(0)
        for k in range(NCH):
            sb, db, _ = bufs[k % 2]
            pend[0].wait()
            pend[1].wait()
            if k + 1 < NCH:
                pend = start(k + 1)

            def step(j, _, sb=sb, db=db):
                b = j * 80
                for q in range(5):
                    sl = pl.ds(b + q * 16, 16)
                    cav = _u(plsc.load_gather(cab, [sb[sl]]))
                    f1 = _mix(cav, 1013904223)
                    plsc.addupdate_scatter(h1b, [db[sl]], _i(f1))
                return _

            lax.fori_loop(0, EC // 80, step, 0)
        pltpu.sync_copy(h1b, hp_out.at[wid])

    sds = jax.ShapeDtypeStruct
    return pl.kernel(
        body,
        out_type=sds((32, HALF), jnp.int32),
        mesh=_MESH,
        scratch_types=[pltpu.VMEM((HALF,), jnp.int32)] * 2
        + [pltpu.VMEM((EC,), jnp.int32)] * 4
        + [pltpu.SemaphoreType.DMA] * 2,
        compiler_params=pltpu.CompilerParams(use_tc_tiling_on_sc=False,
                                             needs_layout_passes=False),
    )


def _stats_phase():
    def body(gs_hbm, cs_hbm, ct_hbm, out_hbm, gsb, vsb, vtb, sb, sem0, sem1):
        c, s, wid = _wid()
        base = c * HALF + s * NPT
        pltpu.sync_copy(gs_hbm.at[pl.ds(base, NPT)], gsb)
        cp0 = pltpu.async_copy(cs_hbm.at[gsb], vsb, sem0)
        cp1 = pltpu.async_copy(ct_hbm.at[gsb], vtb, sem1)
        cp0.wait()
        cp1.wait()

        def step(i, acc):
            b = i * 32
            s0 = pl.ds(b, 16)
            s1 = pl.ds(b + 16, 16)
            return (acc[0] + vsb[s0].astype(jnp.float32),
                    acc[1] + vtb[s0].astype(jnp.float32),
                    acc[2] + vsb[s1].astype(jnp.float32),
                    acc[3] + vtb[s1].astype(jnp.float32))

        z = jnp.zeros((16,), jnp.float32)
        a0, b0, a1, b1 = lax.fori_loop(0, VI // 2, step, (z, z, z, z))
        sb[pl.ds(0, 16)] = a0 + a1
        sb[pl.ds(16, 16)] = b0 + b1
        pltpu.sync_copy(sb, out_hbm.at[wid])

    sds = jax.ShapeDtypeStruct
    return pl.kernel(
        body,
        out_type=sds((32, 32), jnp.float32),
        mesh=_MESH,
        scratch_types=[pltpu.VMEM((NPT,), jnp.int32)] * 3
        + [pltpu.VMEM((32,), jnp.float32)]
        + [pltpu.SemaphoreType.DMA] * 2,
    )


_init_k = _node_phase(init=True)
_node_k = _node_phase(init=False)
_edge_k = _edge_phase()
_stats_k = _stats_phase()

NUM_ROUNDS = 3


def kernel(x_s, edge_index_s, edge_attr_s, x_s_batch,
           x_t, edge_index_t, edge_attr_t, x_t_batch):
    pad = jnp.zeros((HALF - NS,), jnp.int32)
    x = jnp.concatenate([x_s.astype(jnp.int32), pad,
                         x_t.astype(jnp.int32), pad])
    src = jnp.concatenate([edge_index_s[0], edge_index_t[0]]).astype(jnp.int32)
    dst = jnp.concatenate([edge_index_s[1], edge_index_t[1]]).astype(jnp.int32)
    ca, cb, gs, cs, ct = _init_k(x)
    parts = [_stats_k(gs, cs, ct)]
    for _ in range(NUM_ROUNDS):
        hp = _edge_k(ca, src, dst)
        ca, cb, gs, cs, ct = _node_k(ca, cb, hp)
        parts.append(_stats_k(gs, cs, ct))

    p = jnp.stack(parts)
    ssum = jnp.sum(p[:, :16, :16])
    dsum = jnp.sum(p[:, :16, 16:])
    tsum = jnp.sum(p[:, 16:, 16:])
    den = (jnp.maximum(jnp.sqrt(ssum), 1e-8)
           * jnp.maximum(jnp.sqrt(tsum), 1e-8))
    return jnp.reshape(dsum / den, (1,))


# speedup vs baseline: 389.6827x; 1.4918x over previous
import functools

import jax
import jax.numpy as jnp
from jax import lax
from jax.experimental import pallas as pl
from jax.experimental.pallas import tpu as pltpu
from jax.experimental.pallas import tpu_sc as plsc

NS = 50000
NPT = 3200
HALF = 16 * NPT
NP = 2 * HALF
E1 = 1600000
EPT = E1 // 16
EC = 4000
NCH = EPT // EC
VI = NPT // 16
M = 1 << 19
TBL = M + 512
TSL = TBL // 16

_U = jnp.uint32


def _mix(c, seed):
    c = c + _U(seed)
    c = c * _U(0x9E3779B1)
    c = c ^ (c >> _U(15))
    c = c * _U(0x85EBCA6B)
    c = c ^ (c >> _U(13))
    c = c * _U(0xC2B2AE35)
    c = c ^ (c >> _U(16))
    return c


def _u(x):
    return lax.bitcast_convert_type(x, jnp.uint32)


def _i(x):
    return lax.bitcast_convert_type(x, jnp.int32)


_MESH = plsc.VectorSubcoreMesh(core_axis_name="c", subcore_axis_name="s",
                               num_cores=2, num_subcores=16)
_IOTA = lambda: lax.iota(jnp.int32, 16)


def _wid():
    c = lax.axis_index("c")
    s = lax.axis_index("s")
    return c, s, c * 16 + s


def _node_phase(init):
    def body(*refs):
        if init:
            (x_hbm, ca_out, cb_out, gs_out, cs_out, ct_out,
             ab, bb, h1b, tb0, tb1, gsb, ssb, onesb, tbuf, cnt_sp,
             sem0, sem1) = refs
        else:
            (ca_hbm, cb_hbm, hp_hbm, ca_out, cb_out, gs_out,
             cs_out, ct_out, ab, bb, h1b, tb0, tb1, gsb, ssb, onesb, tbuf,
             cnt_sp, sem0, sem1) = refs
        c, s, wid = _wid()
        base = c * HALF + s * NPT

        def zstep(i, _):
            gsb[pl.ds(i * 16, 16)] = jnp.zeros((16,), jnp.int32)
            return _

        lax.fori_loop(0, VI, zstep, 0)
        for k in range(TSL // NPT):
            pltpu.sync_copy(gsb, cnt_sp.at[pl.ds(s * TSL + k * NPT, NPT)])
        rem = TSL % NPT
        if rem:
            pltpu.sync_copy(gsb.at[pl.ds(0, rem)],
                            cnt_sp.at[pl.ds(s * TSL + TSL - rem, rem)])
        if init:
            pltpu.sync_copy(x_hbm.at[pl.ds(base, NPT)], ab)
        else:
            pltpu.sync_copy(ca_hbm.at[pl.ds(base, NPT)], ab)
            pltpu.sync_copy(cb_hbm.at[pl.ds(base, NPT)], bb)
            nsl = pl.ds(s * NPT, NPT)
            pltpu.sync_copy(hp_hbm.at[c * 16, nsl], h1b)
            bufs = ((tb0, sem0), (tb1, sem1))
            pend = pltpu.async_copy(hp_hbm.at[c * 16 + 1, nsl], tb0, sem0)
            for t in range(1, 16):
                tb, _sem = bufs[(t - 1) % 2]
                pend.wait()
                if t + 1 < 16:
                    nb, nsem = bufs[t % 2]
                    pend = pltpu.async_copy(hp_hbm.at[c * 16 + t + 1, nsl],
                                            nb, nsem)

                def astep(i, _, tb=tb):
                    b = i * 64
                    for q in range(4):
                        sl = pl.ds(b + q * 16, 16)
                        h1b[sl] = h1b[sl] + tb[sl]
                    return _

                lax.fori_loop(0, VI // 4, astep, 0)

        def step(i, _):
            sl = pl.ds(i * 16, 16)
            if init:
                xv = _u(ab[sl])
                a = _mix(xv, 0x165667B1)
                b = _mix(xv, 0x85EBCA77)
            else:
                m1 = _mix(_u(h1b[sl]), 0x27D4EB2F)
                a = _mix(_u(ab[sl]) ^ m1, 0x9E3779B9)
                b = _mix(_u(bb[sl]) ^ (m1 * _U(0x165667B1)), 0x68E31DA4)
            slot = _i(_mix(a ^ (b * _U(0x27D4EB2F)), 0xB5297A4D) & _U(M - 1))
            idxl = s * NPT + i * 16 + _IOTA()
            real = idxl < NS
            ssb[sl] = jnp.where(real, slot, M + 256 + wid)
            gsb[sl] = jnp.where(real, slot, M)
            ab[sl] = _i(a)
            bb[sl] = _i(b)
            onesb[sl] = jnp.full((16,), 1, jnp.int32)
            return _

        lax.fori_loop(0, VI, step, 0)
        plsc.subcore_barrier()
        pltpu.sync_copy(onesb, cnt_sp.at[ssb], add=True)
        pltpu.sync_copy(ab, ca_out.at[pl.ds(base, NPT)])
        pltpu.sync_copy(bb, cb_out.at[pl.ds(base, NPT)])
        pltpu.sync_copy(gsb, gs_out.at[pl.ds(base, NPT)])
        plsc.subcore_barrier()
        tsl = pl.ds(s * TSL, TSL)
        pltpu.sync_copy(cnt_sp.at[tsl], tbuf)

        @pl.when(c == 0)
        def _():
            pltpu.sync_copy(tbuf, cs_out.at[tsl])

        @pl.when(c == 1)
        def _():
            pltpu.sync_copy(tbuf, ct_out.at[tsl])

    sds = jax.ShapeDtypeStruct
    return pl.kernel(
        body,
        out_type=[sds((NP,), jnp.int32), sds((NP,), jnp.int32),
                  sds((NP,), jnp.int32), sds((TBL,), jnp.int32),
                  sds((TBL,), jnp.int32)],
        mesh=_MESH,
        scratch_types=[pltpu.VMEM((NPT,), jnp.int32)] * 8
        + [pltpu.VMEM((TSL,), jnp.int32)]
        + [pltpu.VMEM_SHARED((TBL,), jnp.int32)]
        + [pltpu.SemaphoreType.DMA] * 2,
    )


def _edge_phase():
    def body(ca_hbm, src_hbm, dst_hbm, hp_out,
             cab, h1b, srcb0, dstb0, srcb1, dstb1, sem0, sem1):
        c, s, wid = _wid()
        pltpu.sync_copy(ca_hbm.at[pl.ds(c * HALF, HALF)], cab)
        z = jnp.zeros((16,), jnp.int32)

        def zstep(i, _):
            b = i * 64
            h1b[pl.ds(b, 16)] = z
            h1b[pl.ds(b + 16, 16)] = z
            h1b[pl.ds(b + 32, 16)] = z
            h1b[pl.ds(b + 48, 16)] = z
            return _

        lax.fori_loop(0, HALF // 64, zstep, 0)
        ebase = c * E1 + s * EPT
        bufs = ((srcb0, dstb0, sem0), (srcb1, dstb1, sem1))

        def start(k):
            sb, db, sem = bufs[k % 2]
            esl = pl.ds(ebase + k * EC, EC)
            return (pltpu.async_copy(src_hbm.at[esl], sb, sem),
                    pltpu.async_copy(dst_hbm.at[esl], db, sem))

        pend = start(0)
        for k in range(NCH):
            sb, db, _ = bufs[k % 2]
            pend[0].wait()
            pend[1].wait()
            if k + 1 < NCH:
                pend = start(k + 1)

            def step(j, _, sb=sb, db=db):
                b = j * 160
                acc = []
                for q in range(10):
                    sl = pl.ds(b + q * 16, 16)
                    dv = db[sl]
                    cav = _u(plsc.load_gather(cab, [sb[sl]]))
                    acc.append((dv, _i(_mix(cav, 1013904223))))
                for dv, fv in acc:
                    plsc.addupdate_scatter(h1b, [dv], fv)
                return _

            lax.fori_loop(0, EC // 160, step, 0)
        pltpu.sync_copy(h1b, hp_out.at[wid])

    sds = jax.ShapeDtypeStruct
    return pl.kernel(
        body,
        out_type=sds((32, HALF), jnp.int32),
        mesh=_MESH,
        scratch_types=[pltpu.VMEM((HALF,), jnp.int32)] * 2
        + [pltpu.VMEM((EC,), jnp.int32)] * 4
        + [pltpu.SemaphoreType.DMA] * 2,
        compiler_params=pltpu.CompilerParams(use_tc_tiling_on_sc=False,
                                             needs_layout_passes=False),
    )


def _stats_phase():
    def body(gs_hbm, cs_hbm, ct_hbm, out_hbm, gsb, vsb, vtb, sb, sem0, sem1):
        c, s, wid = _wid()
        base = c * HALF + s * NPT
        pltpu.sync_copy(gs_hbm.at[pl.ds(base, NPT)], gsb)
        cp0 = pltpu.async_copy(cs_hbm.at[gsb], vsb, sem0)
        cp1 = pltpu.async_copy(ct_hbm.at[gsb], vtb, sem1)
        cp0.wait()
        cp1.wait()

        def step(i, acc):
            b = i * 32
            s0 = pl.ds(b, 16)
            s1 = pl.ds(b + 16, 16)
            return (acc[0] + vsb[s0].astype(jnp.float32),
                    acc[1] + vtb[s0].astype(jnp.float32),
                    acc[2] + vsb[s1].astype(jnp.float32),
                    acc[3] + vtb[s1].astype(jnp.float32))

        z = jnp.zeros((16,), jnp.float32)
        a0, b0, a1, b1 = lax.fori_loop(0, VI // 2, step, (z, z, z, z))
        sb[pl.ds(0, 16)] = a0 + a1
        sb[pl.ds(16, 16)] = b0 + b1
        pltpu.sync_copy(sb, out_hbm.at[wid])

    sds = jax.ShapeDtypeStruct
    return pl.kernel(
        body,
        out_type=sds((32, 32), jnp.float32),
        mesh=_MESH,
        scratch_types=[pltpu.VMEM((NPT,), jnp.int32)] * 3
        + [pltpu.VMEM((32,), jnp.float32)]
        + [pltpu.SemaphoreType.DMA] * 2,
    )


_init_k = _node_phase(init=True)
_node_k = _node_phase(init=False)
_edge_k = _edge_phase()
_stats_k = _stats_phase()

NUM_ROUNDS = 3


def kernel(x_s, edge_index_s, edge_attr_s, x_s_batch,
           x_t, edge_index_t, edge_attr_t, x_t_batch):
    pad = jnp.zeros((HALF - NS,), jnp.int32)
    x = jnp.concatenate([x_s.astype(jnp.int32), pad,
                         x_t.astype(jnp.int32), pad])
    src = jnp.concatenate([edge_index_s[0], edge_index_t[0]]).astype(jnp.int32)
    dst = jnp.concatenate([edge_index_s[1], edge_index_t[1]]).astype(jnp.int32)
    ca, cb, gs, cs, ct = _init_k(x)
    parts = [_stats_k(gs, cs, ct)]
    for _ in range(NUM_ROUNDS):
        hp = _edge_k(ca, src, dst)
        ca, cb, gs, cs, ct = _node_k(ca, cb, hp)
        parts.append(_stats_k(gs, cs, ct))

    p = jnp.stack(parts)
    ssum = jnp.sum(p[:, :16, :16])
    dsum = jnp.sum(p[:, :16, 16:])
    tsum = jnp.sum(p[:, 16:, 16:])
    den = (jnp.maximum(jnp.sqrt(ssum), 1e-8)
           * jnp.maximum(jnp.sqrt(tsum), 1e-8))
    return jnp.reshape(dsum / den, (1,))


# speedup vs baseline: 402.7222x; 1.0335x over previous
import functools

import jax
import jax.numpy as jnp
from jax import lax
from jax.experimental import pallas as pl
from jax.experimental.pallas import tpu as pltpu
from jax.experimental.pallas import tpu_sc as plsc

NS = 50000
NPT = 3200
HALF = 16 * NPT
NP = 2 * HALF
E1 = 1600000
EPT = E1 // 16
EC = 4000
NCH = EPT // EC
VI = NPT // 16
M = 1 << 19
TBL = M + 512
TSL = TBL // 16

_U = jnp.uint32


def _mix(c, seed):
    c = c + _U(seed)
    c = c * _U(0x9E3779B1)
    c = c ^ (c >> _U(15))
    c = c * _U(0x85EBCA6B)
    c = c ^ (c >> _U(13))
    c = c * _U(0xC2B2AE35)
    c = c ^ (c >> _U(16))
    return c


def _u(x):
    return lax.bitcast_convert_type(x, jnp.uint32)


def _i(x):
    return lax.bitcast_convert_type(x, jnp.int32)


_MESH = plsc.VectorSubcoreMesh(core_axis_name="c", subcore_axis_name="s",
                               num_cores=2, num_subcores=16)
_IOTA = lambda: lax.iota(jnp.int32, 16)


def _wid():
    c = lax.axis_index("c")
    s = lax.axis_index("s")
    return c, s, c * 16 + s


def _node_phase(init):
    def body(*refs):
        if init:
            (x_hbm, ca_out, cb_out, gs_out, cs_out, ct_out,
             ab, bb, h1b, tb0, tb1, gsb, ssb, onesb, tbuf, cnt_sp,
             sem0, sem1) = refs
        else:
            (ca_hbm, cb_hbm, hp_hbm, ca_out, cb_out, gs_out,
             cs_out, ct_out, ab, bb, h1b, tb0, tb1, gsb, ssb, onesb, tbuf,
             cnt_sp, sem0, sem1) = refs
        c, s, wid = _wid()
        base = c * HALF + s * NPT

        def zstep(i, _):
            gsb[pl.ds(i * 16, 16)] = jnp.zeros((16,), jnp.int32)
            return _

        lax.fori_loop(0, VI, zstep, 0)
        for k in range(TSL // NPT):
            pltpu.sync_copy(gsb, cnt_sp.at[pl.ds(s * TSL + k * NPT, NPT)])
        rem = TSL % NPT
        if rem:
            pltpu.sync_copy(gsb.at[pl.ds(0, rem)],
                            cnt_sp.at[pl.ds(s * TSL + TSL - rem, rem)])
        if init:
            pltpu.sync_copy(x_hbm.at[pl.ds(base, NPT)], ab)
        else:
            pltpu.sync_copy(ca_hbm.at[pl.ds(base, NPT)], ab)
            pltpu.sync_copy(cb_hbm.at[pl.ds(base, NPT)], bb)
            nsl = pl.ds(s * NPT, NPT)
            pltpu.sync_copy(hp_hbm.at[c * 16, nsl], h1b)
            bufs = ((tb0, sem0), (tb1, sem1))
            pend = pltpu.async_copy(hp_hbm.at[c * 16 + 1, nsl], tb0, sem0)
            for t in range(1, 16):
                tb, _sem = bufs[(t - 1) % 2]
                pend.wait()
                if t + 1 < 16:
                    nb, nsem = bufs[t % 2]
                    pend = pltpu.async_copy(hp_hbm.at[c * 16 + t + 1, nsl],
                                            nb, nsem)

                def astep(i, _, tb=tb):
                    b = i * 64
                    for q in range(4):
                        sl = pl.ds(b + q * 16, 16)
                        h1b[sl] = h1b[sl] + tb[sl]
                    return _

                lax.fori_loop(0, VI // 4, astep, 0)

        def step(i, _):
            sl = pl.ds(i * 16, 16)
            if init:
                xv = _u(ab[sl])
                a = _mix(xv, 0x165667B1)
                b = _mix(xv, 0x85EBCA77)
            else:
                m1 = _mix(_u(h1b[sl]), 0x27D4EB2F)
                a = _mix(_u(ab[sl]) ^ m1, 0x9E3779B9)
                b = _mix(_u(bb[sl]) ^ (m1 * _U(0x165667B1)), 0x68E31DA4)
            slot = _i(_mix(a ^ (b * _U(0x27D4EB2F)), 0xB5297A4D) & _U(M - 1))
            idxl = s * NPT + i * 16 + _IOTA()
            real = idxl < NS
            ssb[sl] = jnp.where(real, slot, M + 256 + wid)
            gsb[sl] = jnp.where(real, slot, M)
            ab[sl] = _i(a)
            bb[sl] = _i(b)
            onesb[sl] = jnp.full((16,), 1, jnp.int32)
            return _

        lax.fori_loop(0, VI, step, 0)
        plsc.subcore_barrier()
        pltpu.sync_copy(onesb, cnt_sp.at[ssb], add=True)
        pltpu.sync_copy(ab, ca_out.at[pl.ds(base, NPT)])
        pltpu.sync_copy(bb, cb_out.at[pl.ds(base, NPT)])
        pltpu.sync_copy(gsb, gs_out.at[pl.ds(base, NPT)])
        plsc.subcore_barrier()
        tsl = pl.ds(s * TSL, TSL)
        pltpu.sync_copy(cnt_sp.at[tsl], tbuf)

        @pl.when(c == 0)
        def _():
            pltpu.sync_copy(tbuf, cs_out.at[tsl])

        @pl.when(c == 1)
        def _():
            pltpu.sync_copy(tbuf, ct_out.at[tsl])

    sds = jax.ShapeDtypeStruct
    return pl.kernel(
        body,
        out_type=[sds((NP,), jnp.int32), sds((NP,), jnp.int32),
                  sds((NP,), jnp.int32), sds((TBL,), jnp.int32),
                  sds((TBL,), jnp.int32)],
        mesh=_MESH,
        scratch_types=[pltpu.VMEM((NPT,), jnp.int32)] * 8
        + [pltpu.VMEM((TSL,), jnp.int32)]
        + [pltpu.VMEM_SHARED((TBL,), jnp.int32)]
        + [pltpu.SemaphoreType.DMA] * 2,
    )


def _edge_phase():
    def body(ca_hbm, src_hbm, dst_hbm, hp_out,
             cab, h1b, srcb0, dstb0, srcb1, dstb1, sem0, sem1):
        c, s, wid = _wid()
        pltpu.sync_copy(ca_hbm.at[pl.ds(c * HALF, HALF)], cab)
        z = jnp.zeros((16,), jnp.int32)

        def zstep(i, _):
            b = i * 64
            h1b[pl.ds(b, 16)] = z
            h1b[pl.ds(b + 16, 16)] = z
            h1b[pl.ds(b + 32, 16)] = z
            h1b[pl.ds(b + 48, 16)] = z
            return _

        lax.fori_loop(0, HALF // 64, zstep, 0)
        ebase = c * E1 + s * EPT
        bufs = ((srcb0, dstb0, sem0), (srcb1, dstb1, sem1))

        def start(k):
            sb, db, sem = bufs[k % 2]
            esl = pl.ds(ebase + k * EC, EC)
            return (pltpu.async_copy(src_hbm.at[esl], sb, sem),
                    pltpu.async_copy(dst_hbm.at[esl], db, sem))

        pend = start(0)
        for k in range(NCH):
            sb, db, _ = bufs[k % 2]
            pend[0].wait()
            pend[1].wait()
            if k + 1 < NCH:
                pend = start(k + 1)

            def step(j, _, sb=sb, db=db):
                b = j * 160
                acc = []
                for q in range(10):
                    sl = pl.ds(b + q * 16, 16)
                    dv = db[sl]
                    cav = _u(plsc.load_gather(cab, [sb[sl]]))
                    acc.append((dv, _i(_mix(cav, 1013904223))))
                for dv, fv in acc:
                    plsc.addupdate_scatter(h1b, [dv], fv)
                return _

            lax.fori_loop(0, EC // 160, step, 0)
        pltpu.sync_copy(h1b, hp_out.at[wid])

    sds = jax.ShapeDtypeStruct
    return pl.kernel(
        body,
        out_type=sds((32, HALF), jnp.int32),
        mesh=_MESH,
        scratch_types=[pltpu.VMEM((HALF,), jnp.int32)] * 2
        + [pltpu.VMEM((EC,), jnp.int32)] * 4
        + [pltpu.SemaphoreType.DMA] * 2,
        compiler_params=pltpu.CompilerParams(use_tc_tiling_on_sc=False,
                                             needs_layout_passes=False),
    )


def _stats_phase():
    def body(*refs):
        (gs0, gs1, gs2, gs3, cs0, cs1, cs2, cs3, ct0, ct1, ct2, ct3,
         out_hbm, gsb0, gsb1, vs0, vt0, vs1, vt1, sb,
         sem0, sem1, sem2, sem3) = refs
        c, s, wid = _wid()
        base = c * HALF + s * NPT
        gs_refs = (gs0, gs1, gs2, gs3)
        cs_refs = (cs0, cs1, cs2, cs3)
        ct_refs = (ct0, ct1, ct2, ct3)
        bufs = ((gsb0, vs0, vt0, sem0, sem1), (gsb1, vs1, vt1, sem2, sem3))

        def start(L):
            gb, vs, vt, sA, sB = bufs[L % 2]
            pltpu.sync_copy(gs_refs[L].at[pl.ds(base, NPT)], gb)
            return (pltpu.async_copy(cs_refs[L].at[gb], vs, sA),
                    pltpu.async_copy(ct_refs[L].at[gb], vt, sB))

        pend = start(0)
        z = jnp.zeros((16,), jnp.float32)
        acc = (z, z, z, z)
        for L in range(4):
            _gb, vs, vt, _sA, _sB = bufs[L % 2]
            pend[0].wait()
            pend[1].wait()
            if L + 1 < 4:
                pend = start(L + 1)

            def step(i, a, vs=vs, vt=vt):
                b = i * 32
                s0 = pl.ds(b, 16)
                s1 = pl.ds(b + 16, 16)
                return (a[0] + vs[s0].astype(jnp.float32),
                        a[1] + vt[s0].astype(jnp.float32),
                        a[2] + vs[s1].astype(jnp.float32),
                        a[3] + vt[s1].astype(jnp.float32))

            acc = lax.fori_loop(0, VI // 2, step, acc)
        sb[pl.ds(0, 16)] = acc[0] + acc[2]
        sb[pl.ds(16, 16)] = acc[1] + acc[3]
        pltpu.sync_copy(sb, out_hbm.at[wid])

    sds = jax.ShapeDtypeStruct
    return pl.kernel(
        body,
        out_type=sds((32, 32), jnp.float32),
        mesh=_MESH,
        scratch_types=[pltpu.VMEM((NPT,), jnp.int32)] * 6
        + [pltpu.VMEM((32,), jnp.float32)]
        + [pltpu.SemaphoreType.DMA] * 4,
    )


_init_k = _node_phase(init=True)
_node_k = _node_phase(init=False)
_edge_k = _edge_phase()
_stats_k = _stats_phase()

NUM_ROUNDS = 3


def kernel(x_s, edge_index_s, edge_attr_s, x_s_batch,
           x_t, edge_index_t, edge_attr_t, x_t_batch):
    pad = jnp.zeros((HALF - NS,), jnp.int32)
    x = jnp.concatenate([x_s.astype(jnp.int32), pad,
                         x_t.astype(jnp.int32), pad])
    src = jnp.concatenate([edge_index_s[0], edge_index_t[0]]).astype(jnp.int32)
    dst = jnp.concatenate([edge_index_s[1], edge_index_t[1]]).astype(jnp.int32)
    ca, cb, gs, cs, ct = _init_k(x)
    layers = [(gs, cs, ct)]
    for _ in range(NUM_ROUNDS):
        hp = _edge_k(ca, src, dst)
        ca, cb, gs, cs, ct = _node_k(ca, cb, hp)
        layers.append((gs, cs, ct))

    gss, css, cts = zip(*layers)
    p = _stats_k(*gss, *css, *cts)
    ssum = jnp.sum(p[:16, :16])
    dsum = jnp.sum(p[:16, 16:])
    tsum = jnp.sum(p[16:, 16:])
    den = (jnp.maximum(jnp.sqrt(ssum), 1e-8)
           * jnp.maximum(jnp.sqrt(tsum), 1e-8))
    return jnp.reshape(dsum / den, (1,))


# speedup vs baseline: 430.1025x; 1.0680x over previous
import functools

import jax
import jax.numpy as jnp
from jax import lax
from jax.experimental import pallas as pl
from jax.experimental.pallas import tpu as pltpu
from jax.experimental.pallas import tpu_sc as plsc

NS = 50000
NPT = 3200
HALF = 16 * NPT
NP = 2 * HALF
E1 = 1600000
EPT = E1 // 16
EC = 4000
NCH = EPT // EC
VI = NPT // 16
M = 1 << 19
TBL = M + 512
TSL = TBL // 16

_U = jnp.uint32


def _mix(c, seed):
    c = c + _U(seed)
    c = c * _U(0x9E3779B1)
    c = c ^ (c >> _U(15))
    c = c * _U(0x85EBCA6B)
    c = c ^ (c >> _U(13))
    c = c * _U(0xC2B2AE35)
    c = c ^ (c >> _U(16))
    return c


def _u(x):
    return lax.bitcast_convert_type(x, jnp.uint32)


def _i(x):
    return lax.bitcast_convert_type(x, jnp.int32)


_MESH = plsc.VectorSubcoreMesh(core_axis_name="c", subcore_axis_name="s",
                               num_cores=2, num_subcores=16)
_IOTA = lambda: lax.iota(jnp.int32, 16)


def _wid():
    c = lax.axis_index("c")
    s = lax.axis_index("s")
    return c, s, c * 16 + s


def _node_phase(init):
    def body(*refs):
        if init:
            (x_hbm, ca_out, cb_out, gs_out, cs_out, ct_out,
             ab, bb, h1b, tb0, tb1, gsb, ssb, onesb, tbuf, cnt_sp,
             sem0, sem1) = refs
        else:
            (ca_hbm, cb_hbm, hp_hbm, ca_out, cb_out, gs_out,
             cs_out, ct_out, ab, bb, h1b, tb0, tb1, gsb, ssb, onesb, tbuf,
             cnt_sp, sem0, sem1) = refs
        c, s, wid = _wid()
        base = c * HALF + s * NPT

        def zstep(i, _):
            gsb[pl.ds(i * 16, 16)] = jnp.zeros((16,), jnp.int32)
            return _

        lax.fori_loop(0, VI, zstep, 0)
        for k in range(TSL // NPT):
            pltpu.sync_copy(gsb, cnt_sp.at[pl.ds(s * TSL + k * NPT, NPT)])
        rem = TSL % NPT
        if rem:
            pltpu.sync_copy(gsb.at[pl.ds(0, rem)],
                            cnt_sp.at[pl.ds(s * TSL + TSL - rem, rem)])
        if init:
            pltpu.sync_copy(x_hbm.at[pl.ds(base, NPT)], ab)
        else:
            pltpu.sync_copy(ca_hbm.at[pl.ds(base, NPT)], ab)
            pltpu.sync_copy(cb_hbm.at[pl.ds(base, NPT)], bb)
            nsl = pl.ds(s * NPT, NPT)
            pltpu.sync_copy(hp_hbm.at[c * 16, nsl], h1b)
            bufs = ((tb0, sem0), (tb1, sem1))
            pend = pltpu.async_copy(hp_hbm.at[c * 16 + 1, nsl], tb0, sem0)
            for t in range(1, 16):
                tb, _sem = bufs[(t - 1) % 2]
                pend.wait()
                if t + 1 < 16:
                    nb, nsem = bufs[t % 2]
                    pend = pltpu.async_copy(hp_hbm.at[c * 16 + t + 1, nsl],
                                            nb, nsem)

                def astep(i, _, tb=tb):
                    b = i * 64
                    for q in range(4):
                        sl = pl.ds(b + q * 16, 16)
                        h1b[sl] = h1b[sl] + tb[sl]
                    return _

                lax.fori_loop(0, VI // 4, astep, 0)

        def step(i, _):
            sl = pl.ds(i * 16, 16)
            if init:
                xv = _u(ab[sl])
                a = _mix(xv, 0x165667B1)
                b = _mix(xv, 0x85EBCA77)
            else:
                m1 = _mix(_u(h1b[sl]), 0x27D4EB2F)
                a = _mix(_u(ab[sl]) ^ m1, 0x9E3779B9)
                b = _mix(_u(bb[sl]) ^ (m1 * _U(0x165667B1)), 0x68E31DA4)
            slot = _i(_mix(a ^ (b * _U(0x27D4EB2F)), 0xB5297A4D) & _U(M - 1))
            idxl = s * NPT + i * 16 + _IOTA()
            real = idxl < NS
            ssb[sl] = jnp.where(real, slot, M + 256 + wid)
            gsb[sl] = jnp.where(real, slot, M)
            ab[sl] = _i(a)
            bb[sl] = _i(b)
            onesb[sl] = jnp.full((16,), 1, jnp.int32)
            return _

        lax.fori_loop(0, VI, step, 0)
        plsc.subcore_barrier()
        pltpu.sync_copy(onesb, cnt_sp.at[ssb], add=True)
        pltpu.sync_copy(ab, ca_out.at[pl.ds(base, NPT)])
        pltpu.sync_copy(bb, cb_out.at[pl.ds(base, NPT)])
        pltpu.sync_copy(gsb, gs_out.at[pl.ds(base, NPT)])
        plsc.subcore_barrier()
        tsl = pl.ds(s * TSL, TSL)
        pltpu.sync_copy(cnt_sp.at[tsl], tbuf)

        @pl.when(c == 0)
        def _():
            pltpu.sync_copy(tbuf, cs_out.at[tsl])

        @pl.when(c == 1)
        def _():
            pltpu.sync_copy(tbuf, ct_out.at[tsl])

    sds = jax.ShapeDtypeStruct
    return pl.kernel(
        body,
        out_type=[sds((NP,), jnp.int32), sds((NP,), jnp.int32),
                  sds((NP,), jnp.int32), sds((TBL,), jnp.int32),
                  sds((TBL,), jnp.int32)],
        mesh=_MESH,
        scratch_types=[pltpu.VMEM((NPT,), jnp.int32)] * 8
        + [pltpu.VMEM((TSL,), jnp.int32)]
        + [pltpu.VMEM_SHARED((TBL,), jnp.int32)]
        + [pltpu.SemaphoreType.DMA] * 2,
    )


def _edge_phase():
    def body(ca_hbm, src_hbm, dst_hbm, hp_out,
             cab, h1b, srcb0, dstb0, srcb1, dstb1, sem0, sem1, semc):
        c, s, wid = _wid()
        cacp = pltpu.async_copy(ca_hbm.at[pl.ds(c * HALF, HALF)], cab, semc)
        ebase = c * E1 + s * EPT
        bufs = ((srcb0, dstb0, sem0), (srcb1, dstb1, sem1))

        def start(k):
            sb, db, sem = bufs[k % 2]
            esl = pl.ds(ebase + k * EC, EC)
            return (pltpu.async_copy(src_hbm.at[esl], sb, sem),
                    pltpu.async_copy(dst_hbm.at[esl], db, sem))

        pend = start(0)
        z = jnp.zeros((16,), jnp.int32)

        def zstep(i, _):
            b = i * 64
            h1b[pl.ds(b, 16)] = z
            h1b[pl.ds(b + 16, 16)] = z
            h1b[pl.ds(b + 32, 16)] = z
            h1b[pl.ds(b + 48, 16)] = z
            return _

        lax.fori_loop(0, HALF // 64, zstep, 0)
        cacp.wait()
        for k in range(NCH):
            sb, db, _ = bufs[k % 2]
            pend[0].wait()
            pend[1].wait()
            if k + 1 < NCH:
                pend = start(k + 1)

            def step(j, _, sb=sb, db=db):
                b = j * 160
                acc = []
                for q in range(10):
                    sl = pl.ds(b + q * 16, 16)
                    dv = db[sl]
                    cav = _u(plsc.load_gather(cab, [sb[sl]]))
                    acc.append((dv, _i(_mix(cav, 1013904223))))
                for dv, fv in acc:
                    plsc.addupdate_scatter(h1b, [dv], fv)
                return _

            lax.fori_loop(0, EC // 160, step, 0)
        pltpu.sync_copy(h1b, hp_out.at[wid])

    sds = jax.ShapeDtypeStruct
    return pl.kernel(
        body,
        out_type=sds((32, HALF), jnp.int32),
        mesh=_MESH,
        scratch_types=[pltpu.VMEM((HALF,), jnp.int32)] * 2
        + [pltpu.VMEM((EC,), jnp.int32)] * 4
        + [pltpu.SemaphoreType.DMA] * 3,
        compiler_params=pltpu.CompilerParams(use_tc_tiling_on_sc=False,
                                             needs_layout_passes=False),
    )


def _stats_phase():
    def body(*refs):
        (gs0, gs1, gs2, gs3, cs0, cs1, cs2, cs3, ct0, ct1, ct2, ct3,
         out_hbm, sb) = refs[:14]
        gbufs = refs[14:18]
        vbufs = refs[18:26]
        sems = refs[26:34]
        c, s, wid = _wid()
        base = c * HALF + s * NPT
        gs_refs = (gs0, gs1, gs2, gs3)
        cs_refs = (cs0, cs1, cs2, cs3)
        ct_refs = (ct0, ct1, ct2, ct3)

        for L in range(4):
            pltpu.sync_copy(gs_refs[L].at[pl.ds(base, NPT)], gbufs[L])
        pend = []
        for L in range(4):
            pend.append(pltpu.async_copy(cs_refs[L].at[gbufs[L]],
                                         vbufs[2 * L], sems[2 * L]))
            pend.append(pltpu.async_copy(ct_refs[L].at[gbufs[L]],
                                         vbufs[2 * L + 1], sems[2 * L + 1]))
        z = jnp.zeros((16,), jnp.float32)
        acc = (z, z, z, z)
        for L in range(4):
            vs, vt = vbufs[2 * L], vbufs[2 * L + 1]
            pend[2 * L].wait()
            pend[2 * L + 1].wait()

            def step(i, a, vs=vs, vt=vt):
                b = i * 32
                s0 = pl.ds(b, 16)
                s1 = pl.ds(b + 16, 16)
                return (a[0] + vs[s0].astype(jnp.float32),
                        a[1] + vt[s0].astype(jnp.float32),
                        a[2] + vs[s1].astype(jnp.float32),
                        a[3] + vt[s1].astype(jnp.float32))

            acc = lax.fori_loop(0, VI // 2, step, acc)
        sb[pl.ds(0, 16)] = acc[0] + acc[2]
        sb[pl.ds(16, 16)] = acc[1] + acc[3]
        pltpu.sync_copy(sb, out_hbm.at[wid])

    sds = jax.ShapeDtypeStruct
    return pl.kernel(
        body,
        out_type=sds((32, 32), jnp.float32),
        mesh=_MESH,
        scratch_types=[pltpu.VMEM((32,), jnp.float32)]
        + [pltpu.VMEM((NPT,), jnp.int32)] * 12
        + [pltpu.SemaphoreType.DMA] * 8,
    )


_init_k = _node_phase(init=True)
_node_k = _node_phase(init=False)
_edge_k = _edge_phase()
_stats_k = _stats_phase()

NUM_ROUNDS = 3


def kernel(x_s, edge_index_s, edge_attr_s, x_s_batch,
           x_t, edge_index_t, edge_attr_t, x_t_batch):
    pad = jnp.zeros((HALF - NS,), jnp.int32)
    x = jnp.concatenate([x_s.astype(jnp.int32), pad,
                         x_t.astype(jnp.int32), pad])
    src = jnp.concatenate([edge_index_s[0], edge_index_t[0]]).astype(jnp.int32)
    dst = jnp.concatenate([edge_index_s[1], edge_index_t[1]]).astype(jnp.int32)
    ca, cb, gs, cs, ct = _init_k(x)
    layers = [(gs, cs, ct)]
    for _ in range(NUM_ROUNDS):
        hp = _edge_k(ca, src, dst)
        ca, cb, gs, cs, ct = _node_k(ca, cb, hp)
        layers.append((gs, cs, ct))

    gss, css, cts = zip(*layers)
    p = _stats_k(*gss, *css, *cts)
    ssum = jnp.sum(p[:16, :16])
    dsum = jnp.sum(p[:16, 16:])
    tsum = jnp.sum(p[16:, 16:])
    den = (jnp.maximum(jnp.sqrt(ssum), 1e-8)
           * jnp.maximum(jnp.sqrt(tsum), 1e-8))
    return jnp.reshape(dsum / den, (1,))
